# baseline (speedup 1.0000x reference)
import functools

import jax
import jax.numpy as jnp
from jax import lax
from jax.experimental import pallas as pl
from jax.experimental.pallas import tpu as pltpu
from jax.experimental.pallas import tpu_sc as plsc

NC = 2
NS = 16
NW = NC * NS
LANE = 128


def _sc_mesh():
    return plsc.VectorSubcoreMesh(
        core_axis_name="c", subcore_axis_name="s", num_cores=NC, num_subcores=NS
    )


def _row_chunks(total, step):
    out = []
    base = 0
    while base < total:
        out.append((base, min(step, total - base)))
        base += step
    return out


def _propagate_partials(y, col3, row3, n2, ch):
    c_feat = y.shape[1]
    rpt = n2 // NS
    chunks = _row_chunks(rpt, LANE)

    @functools.partial(
        pl.kernel,
        out_type=jax.ShapeDtypeStruct((NC, n2, c_feat), jnp.float32),
        mesh=_sc_mesh(),
        scratch_types=[
            pltpu.VMEM((ch, LANE), jnp.int32),
            pltpu.VMEM((ch, LANE), jnp.int32),
            pltpu.VMEM((LANE, c_feat), jnp.float32),
            pltpu.VMEM_SHARED((n2, c_feat), jnp.float32),
        ],
    )
    def body(y_hbm, col_hbm, row_hbm, out_hbm, col_v, row_v, gbuf, acc_sh):
        c = lax.axis_index("c")
        s = lax.axis_index("s")
        wid = c * NS + s
        for off, sz in chunks:
            base = s * rpt + off
            pltpu.sync_copy(y_hbm.at[pl.ds(base, sz)], gbuf.at[pl.ds(0, sz)])
            pltpu.sync_copy(gbuf.at[pl.ds(0, sz)], acc_sh.at[pl.ds(base, sz)])
        pltpu.sync_copy(col_hbm.at[wid], col_v)
        pltpu.sync_copy(row_hbm.at[wid], row_v)
        plsc.subcore_barrier()

        def step(j, carry):
            pltpu.sync_copy(y_hbm.at[col_v.at[j]], gbuf)
            pltpu.sync_copy(gbuf, acc_sh.at[row_v.at[j]], add=True)
            return carry

        lax.fori_loop(0, ch, step, 0)
        plsc.subcore_barrier()
        for off, sz in chunks:
            base = s * rpt + off
            pltpu.sync_copy(acc_sh.at[pl.ds(base, sz)], gbuf.at[pl.ds(0, sz)])
            pltpu.sync_copy(gbuf.at[pl.ds(0, sz)], out_hbm.at[c, pl.ds(base, sz)])

    return body(y, col3, row3)


def _scale_body(d_ref, x_ref, y_ref, dv_ref):
    deg = d_ref[...] - 1.0
    dinv = lax.rsqrt(deg)
    dv_ref[...] = dinv
    y_ref[...] = dinv * x_ref[...]


def _final_body(dv_ref, y_ref, q0_ref, q1_ref, w_ref, b_ref, o_ref):
    a = dv_ref[...] * (q0_ref[...] + q1_ref[...] - y_ref[...])
    o_ref[...] = (
        jnp.dot(a, w_ref[...], preferred_element_type=jnp.float32) + b_ref[...]
    )


def kernel(x, edge_index, W, b):
    n, c_feat = x.shape
    e = edge_index.shape[1]
    n2 = -(-(n + 1) // LANE) * LANE
    ch = -(-e // (NW * LANE))
    pad = NW * ch * LANE - e

    row = edge_index[0]
    col = edge_index[1]
    row_p = jnp.concatenate([row, jnp.full((pad,), n, dtype=jnp.int32)])
    col_p = jnp.concatenate([col, jnp.zeros((pad,), dtype=jnp.int32)])
    row3 = row_p.reshape(NW, ch, LANE)
    col3 = col_p.reshape(NW, ch, LANE)
    x_pad = jnp.concatenate(
        [x, jnp.zeros((n2 - n, c_feat), jnp.float32)], axis=0)

    ones_tab = jnp.ones((n2, c_feat), jnp.float32)
    col3z = jnp.zeros_like(col3)
    qd = _propagate_partials(ones_tab, col3z, row3, n2, ch)
    d2 = qd[0, :, 0:1] + qd[1, :, 0:1]

    blk = LANE
    nblk = n2 // blk
    y, dv = pl.pallas_call(
        _scale_body,
        grid=(nblk,),
        in_specs=[
            pl.BlockSpec((blk, 1), lambda i: (i, 0)),
            pl.BlockSpec((blk, c_feat), lambda i: (i, 0)),
        ],
        out_specs=[
            pl.BlockSpec((blk, c_feat), lambda i: (i, 0)),
            pl.BlockSpec((blk, 1), lambda i: (i, 0)),
        ],
        out_shape=[
            jax.ShapeDtypeStruct((n2, c_feat), jnp.float32),
            jax.ShapeDtypeStruct((n2, 1), jnp.float32),
        ],
    )(d2, x_pad)

    q = _propagate_partials(y, col3, row3, n2, ch)

    out = pl.pallas_call(
        _final_body,
        grid=(nblk,),
        in_specs=[
            pl.BlockSpec((blk, 1), lambda i: (i, 0)),
            pl.BlockSpec((blk, c_feat), lambda i: (i, 0)),
            pl.BlockSpec((blk, c_feat), lambda i: (i, 0)),
            pl.BlockSpec((blk, c_feat), lambda i: (i, 0)),
            pl.BlockSpec((c_feat, c_feat), lambda i: (0, 0)),
            pl.BlockSpec((1, c_feat), lambda i: (0, 0)),
        ],
        out_specs=pl.BlockSpec((blk, c_feat), lambda i: (i, 0)),
        out_shape=jax.ShapeDtypeStruct((n2, c_feat), jnp.float32),
    )(dv, y, q[0], q[1], W, b.reshape(1, c_feat))
    return out[:n]


# speedup vs baseline: 23.2286x; 23.2286x over previous
import functools

import jax
import jax.numpy as jnp
from jax import lax
from jax.experimental import pallas as pl
from jax.experimental.pallas import tpu as pltpu
from jax.experimental.pallas import tpu_sc as plsc

NC = 2
NS = 16
NW = NC * NS
LANE = 128


def _sc_mesh():
    return plsc.VectorSubcoreMesh(
        core_axis_name="c", subcore_axis_name="s", num_cores=NC, num_subcores=NS
    )


def _row_chunks(total, step):
    out = []
    base = 0
    while base < total:
        out.append((base, min(step, total - base)))
        base += step
    return out


def _propagate_partials(y, col3, row3, n2, ch):
    c_feat = y.shape[1]
    rpt = n2 // NS
    chunks = _row_chunks(rpt, LANE)

    @functools.partial(
        pl.kernel,
        out_type=jax.ShapeDtypeStruct((NC, n2, c_feat), jnp.float32),
        mesh=_sc_mesh(),
        scratch_types=[
            pltpu.VMEM((ch, LANE), jnp.int32),
            pltpu.VMEM((ch, LANE), jnp.int32),
            pltpu.VMEM((LANE, c_feat), jnp.float32),
            pltpu.VMEM_SHARED((n2, c_feat), jnp.float32),
        ],
    )
    def body(y_hbm, col_hbm, row_hbm, out_hbm, col_v, row_v, gbuf, acc_sh):
        c = lax.axis_index("c")
        s = lax.axis_index("s")
        wid = c * NS + s
        for off, sz in chunks:
            base = s * rpt + off
            pltpu.sync_copy(y_hbm.at[pl.ds(base, sz)], gbuf.at[pl.ds(0, sz)])
            pltpu.sync_copy(gbuf.at[pl.ds(0, sz)], acc_sh.at[pl.ds(base, sz)])
        pltpu.sync_copy(col_hbm.at[wid], col_v)
        pltpu.sync_copy(row_hbm.at[wid], row_v)
        plsc.subcore_barrier()

        def step(j, carry):
            pltpu.sync_copy(y_hbm.at[col_v.at[j]], gbuf)
            pltpu.sync_copy(gbuf, acc_sh.at[row_v.at[j]], add=True)
            return carry

        lax.fori_loop(0, ch, step, 0)
        plsc.subcore_barrier()
        for off, sz in chunks:
            base = s * rpt + off
            pltpu.sync_copy(acc_sh.at[pl.ds(base, sz)], gbuf.at[pl.ds(0, sz)])
            pltpu.sync_copy(gbuf.at[pl.ds(0, sz)], out_hbm.at[c, pl.ds(base, sz)])

    return body(y, col3, row3)


def _count_partials(ones_tab, row3, n2, ch):
    c_feat = ones_tab.shape[1]
    rpt = n2 // NS
    chunks = _row_chunks(rpt, LANE)

    @functools.partial(
        pl.kernel,
        out_type=jax.ShapeDtypeStruct((NC, n2, c_feat), jnp.float32),
        mesh=_sc_mesh(),
        scratch_types=[
            pltpu.VMEM((ch, LANE), jnp.int32),
            pltpu.VMEM((LANE, c_feat), jnp.float32),
            pltpu.VMEM_SHARED((n2, c_feat), jnp.float32),
        ],
    )
    def body(ones_hbm, row_hbm, out_hbm, row_v, gbuf, acc_sh):
        c = lax.axis_index("c")
        s = lax.axis_index("s")
        wid = c * NS + s
        for off, sz in chunks:
            base = s * rpt + off
            pltpu.sync_copy(ones_hbm.at[pl.ds(base, sz)], gbuf.at[pl.ds(0, sz)])
            pltpu.sync_copy(gbuf.at[pl.ds(0, sz)], acc_sh.at[pl.ds(base, sz)])
        pltpu.sync_copy(ones_hbm.at[pl.ds(0, LANE)], gbuf)
        pltpu.sync_copy(row_hbm.at[wid], row_v)
        plsc.subcore_barrier()

        def step(j, carry):
            pltpu.sync_copy(gbuf, acc_sh.at[row_v.at[j]], add=True)
            return carry

        lax.fori_loop(0, ch, step, 0)
        plsc.subcore_barrier()
        for off, sz in chunks:
            base = s * rpt + off
            pltpu.sync_copy(acc_sh.at[pl.ds(base, sz)], gbuf.at[pl.ds(0, sz)])
            pltpu.sync_copy(gbuf.at[pl.ds(0, sz)], out_hbm.at[c, pl.ds(base, sz)])

    return body(ones_tab, row3)


def _scale_body(d_ref, x_ref, y_ref, dv_ref):
    deg = d_ref[...] - 1.0
    dinv = lax.rsqrt(deg)
    dv_ref[...] = dinv
    y_ref[...] = dinv * x_ref[...]


def _final_body(dv_ref, y_ref, q0_ref, q1_ref, w_ref, b_ref, o_ref):
    a = dv_ref[...] * (q0_ref[...] + q1_ref[...] - y_ref[...])
    o_ref[...] = (
        jnp.dot(a, w_ref[...], preferred_element_type=jnp.float32) + b_ref[...]
    )


def kernel(x, edge_index, W, b):
    n, c_feat = x.shape
    e = edge_index.shape[1]
    n2 = -(-(n + 1) // LANE) * LANE
    ch = -(-e // (NW * LANE))
    pad = NW * ch * LANE - e

    row = edge_index[0]
    col = edge_index[1]
    row_p = jnp.concatenate([row, jnp.full((pad,), n, dtype=jnp.int32)])
    col_p = jnp.concatenate([col, jnp.zeros((pad,), dtype=jnp.int32)])
    row3 = row_p.reshape(NW, ch, LANE)
    col3 = col_p.reshape(NW, ch, LANE)
    x_pad = jnp.concatenate(
        [x, jnp.zeros((n2 - n, c_feat), jnp.float32)], axis=0)

    ones_tab = jnp.ones((n2, c_feat), jnp.float32)
    qd = _count_partials(ones_tab, row3, n2, ch)
    d2 = qd[0, :, 0:1] + qd[1, :, 0:1]

    blk = LANE
    nblk = n2 // blk
    y, dv = pl.pallas_call(
        _scale_body,
        grid=(nblk,),
        in_specs=[
            pl.BlockSpec((blk, 1), lambda i: (i, 0)),
            pl.BlockSpec((blk, c_feat), lambda i: (i, 0)),
        ],
        out_specs=[
            pl.BlockSpec((blk, c_feat), lambda i: (i, 0)),
            pl.BlockSpec((blk, 1), lambda i: (i, 0)),
        ],
        out_shape=[
            jax.ShapeDtypeStruct((n2, c_feat), jnp.float32),
            jax.ShapeDtypeStruct((n2, 1), jnp.float32),
        ],
    )(d2, x_pad)

    q = _propagate_partials(y, col3, row3, n2, ch)

    out = pl.pallas_call(
        _final_body,
        grid=(nblk,),
        in_specs=[
            pl.BlockSpec((blk, 1), lambda i: (i, 0)),
            pl.BlockSpec((blk, c_feat), lambda i: (i, 0)),
            pl.BlockSpec((blk, c_feat), lambda i: (i, 0)),
            pl.BlockSpec((blk, c_feat), lambda i: (i, 0)),
            pl.BlockSpec((c_feat, c_feat), lambda i: (0, 0)),
            pl.BlockSpec((1, c_feat), lambda i: (0, 0)),
        ],
        out_specs=pl.BlockSpec((blk, c_feat), lambda i: (i, 0)),
        out_shape=jax.ShapeDtypeStruct((n2, c_feat), jnp.float32),
    )(dv, y, q[0], q[1], W, b.reshape(1, c_feat))
    return out[:n]
